# baseline (speedup 1.0000x reference)
import functools
import math

import numpy as np
import jax
import jax.numpy as jnp
from jax.experimental import pallas as pl
from jax.experimental.pallas import tpu as pltpu

_FACTOR = 5
_NEG = np.float32(-3.4e38)


def _threefry2x32_np(k1, k2, x0, x1):
    rot = (13, 15, 26, 6, 17, 29, 16, 24)

    def rotl(x, d):
        return (x << np.uint32(d)) | (x >> np.uint32(32 - d))

    ks = [np.uint32(k1), np.uint32(k2),
          np.uint32(np.uint32(k1) ^ np.uint32(k2) ^ np.uint32(0x1BD11BDA))]
    x = [x0 + ks[0], x1 + ks[1]]
    ksched = [(ks[1], ks[2]), (ks[2], ks[0]), (ks[0], ks[1]),
              (ks[1], ks[2]), (ks[2], ks[0])]
    for i in range(5):
        for r in rot[:4] if i % 2 == 0 else rot[4:]:
            x[0] = x[0] + x[1]
            x[1] = rotl(x[1], r)
            x[1] = x[0] ^ x[1]
        a, b = ksched[i]
        x[0] = x[0] + a
        x[1] = x[1] + b + np.uint32(i + 1)
    return x[0], x[1]


def _fixed_sample_indices(L_Q: int, L_K: int, U_part: int) -> np.ndarray:
    with np.errstate(over="ignore"):
        b1, b2 = _threefry2x32_np(np.uint32(0), np.uint32(42),
                                  np.zeros(2, np.uint32),
                                  np.arange(2, dtype=np.uint32))
        n = L_Q * U_part
        hi_cnt = np.zeros(n, np.uint32)
        lo_cnt = np.arange(n, dtype=np.uint32)
        ha, hb = _threefry2x32_np(b1[0], b2[0], hi_cnt, lo_cnt)
        la, lb = _threefry2x32_np(b1[1], b2[1], hi_cnt, lo_cnt)
        higher_bits, lower_bits = ha ^ hb, la ^ lb
        span = np.uint32(L_K)
        mult = np.uint32((2 ** 16) % L_K)
        mult = np.uint32((int(mult) * int(mult)) % L_K)
        offset = ((higher_bits % span) * mult + lower_bits % span) % span
    return offset.astype(np.int64).reshape(L_Q, U_part)


@functools.lru_cache(maxsize=None)
def _sample_count_matrix_T(L_Q: int, L_K: int, U_part: int):
    idx_np = _fixed_sample_indices(L_Q, L_K, U_part)
    ct = np.zeros((L_K, L_Q), np.float32)
    np.add.at(ct, (idx_np.reshape(-1), np.repeat(np.arange(L_Q), U_part)), 1.0)
    return np.where(ct > 0, ct, _NEG).astype(np.float32)


_sample_count_matrix_T(2048, 2048, 40)


def _body(u, u_pad, L_K, D, scale, q_ref, k_ref, v_ref, ct_ref, o_ref):
    L_Q = q_ref.shape[1]
    nh = q_ref.shape[2] // D
    Qs = [q_ref[0, :, h * D:(h + 1) * D] for h in range(nh)]
    Ks = [k_ref[0, :, h * D:(h + 1) * D] for h in range(nh)]
    Vs = [v_ref[0, :, h * D:(h + 1) * D] for h in range(nh)]

    CH = min(512, L_Q)
    n_ch = L_Q // CH
    rows = [[] for _ in range(nh)]
    for ci in range(n_ch):
        g = ct_ref[:, ci * CH:(ci + 1) * CH]
        cct = jnp.maximum(g, np.float32(0.0))
        ma = jnp.minimum(g, np.float32(0.0))
        for h in range(nh):
            qc = Qs[h][ci * CH:(ci + 1) * CH, :]
            qkt = jax.lax.dot_general(Ks[h], qc, (((1,), (1,)), ((), ())),
                                      preferred_element_type=jnp.float32)
            smax = jnp.max(qkt + ma, axis=0)
            ssum = jnp.sum(qkt * cct, axis=0)
            m = smax - ssum * np.float32(1.0 / L_K)
            rows[h].append(m.reshape(1, CH))
    for h in range(nh):
        _one_head(u, u_pad, L_K, D, scale, h, rows[h], Qs[h], Ks[h], Vs[h],
                  o_ref)


def _one_head(u, u_pad, L_K, D, scale, h, rows, Q, K, V, o_ref):
    L_Q = o_ref.shape[1]
    n_ch = len(rows)
    M = jnp.concatenate(rows, axis=1) if n_ch > 1 else rows[0]

    lane_idx = jax.lax.broadcasted_iota(jnp.int32, (1, L_Q), 1)
    rank = jnp.zeros((1, L_Q), jnp.float32)
    RC = min(512, L_Q)
    ones8r = jnp.full((8, RC), np.float32(1.0), jnp.float32)
    eye = (jax.lax.broadcasted_iota(jnp.int32, (RC, RC), 0)
           == jax.lax.broadcasted_iota(jnp.int32, (RC, RC), 1)
           ).astype(jnp.float32)
    for rc in range(L_Q // RC):
        mrow = jax.lax.dot_general(
            eye, M[:, rc * RC:(rc + 1) * RC], (((1,), (1,)), ((), ())),
            preferred_element_type=jnp.float32)
        rowi = (jax.lax.broadcasted_iota(jnp.int32, (RC, 1), 0)
                + np.int32(rc * RC))
        gt = mrow > M
        eq = mrow == M
        tri = rowi < lane_idx
        cond = jnp.logical_or(gt, jnp.logical_and(eq, tri))
        condf = jnp.where(cond, np.float32(1.0), np.float32(0.0))
        rank = rank + jnp.sum(condf, axis=0, keepdims=True)

    ranki = rank.astype(jnp.int32)
    uio = jax.lax.broadcasted_iota(jnp.int32, (u_pad, 1), 0)
    R = ((ranki == uio) & (uio < np.int32(u))).astype(jnp.float32)

    Qr = jax.lax.dot_general(R, Q, (((1,), (0,)), ((), ())),
                             preferred_element_type=jnp.float32)
    scores = jax.lax.dot_general(Qr, K, (((1,), (1,)), ((), ())),
                                 preferred_element_type=jnp.float32)
    scores = scores * np.float32(scale)
    smax2 = jnp.max(scores, axis=1, keepdims=True)
    e = jnp.exp(scores - smax2)
    attn = e / jnp.sum(e, axis=1, keepdims=True)
    upd = jax.lax.dot_general(attn, V, (((1,), (0,)), ((), ())),
                              preferred_element_type=jnp.float32)
    mean_v = jnp.sum(V, axis=0, keepdims=True) * np.float32(1.0 / L_K)
    delta = upd - mean_v
    ctx = jax.lax.dot_general(R, delta, (((0,), (0,)), ((), ())),
                              preferred_element_type=jnp.float32)
    o_ref[0, :, h * D:(h + 1) * D] = ctx + mean_v


def kernel(queries, keys, values):
    B, L_Q, H, D = queries.shape
    L_K = keys.shape[1]
    U_part = min(_FACTOR * int(np.ceil(np.log(L_K))), L_K)
    u = min(_FACTOR * int(np.ceil(np.log(L_Q))), L_Q)
    u_pad = -(-u // 8) * 8
    scale = 1.0 / math.sqrt(D)
    ct = _sample_count_matrix_T(L_Q, L_K, U_part)

    HPB = max(1, 128 // D)
    ng = H // HPB
    q3 = queries.reshape(B, L_Q, H * D)
    k3 = keys.reshape(B, L_K, H * D)
    v3 = values.reshape(B, L_K, H * D)
    bh = pl.BlockSpec((1, L_Q, HPB * D), lambda i: (i // ng, 0, i % ng))
    out = pl.pallas_call(
        functools.partial(_body, u, u_pad, L_K, D, scale),
        grid=(B * ng,),
        in_specs=[
            bh,
            bh,
            bh,
            pl.BlockSpec((L_K, L_Q), lambda i: (0, 0)),
        ],
        out_specs=bh,
        out_shape=jax.ShapeDtypeStruct((B, L_Q, H * D), jnp.float32),
        compiler_params=pltpu.CompilerParams(
            dimension_semantics=("arbitrary",),
        ),
    )(q3, k3, v3, ct)

    return out.reshape(B, L_Q, H, D)


# speedup vs baseline: 1.0939x; 1.0939x over previous
import functools
import math

import numpy as np
import jax
import jax.numpy as jnp
from jax.experimental import pallas as pl
from jax.experimental.pallas import tpu as pltpu

_FACTOR = 5
_NEG = np.float32(-3.4e38)


def _threefry2x32_np(k1, k2, x0, x1):
    rot = (13, 15, 26, 6, 17, 29, 16, 24)

    def rotl(x, d):
        return (x << np.uint32(d)) | (x >> np.uint32(32 - d))

    ks = [np.uint32(k1), np.uint32(k2),
          np.uint32(np.uint32(k1) ^ np.uint32(k2) ^ np.uint32(0x1BD11BDA))]
    x = [x0 + ks[0], x1 + ks[1]]
    ksched = [(ks[1], ks[2]), (ks[2], ks[0]), (ks[0], ks[1]),
              (ks[1], ks[2]), (ks[2], ks[0])]
    for i in range(5):
        for r in rot[:4] if i % 2 == 0 else rot[4:]:
            x[0] = x[0] + x[1]
            x[1] = rotl(x[1], r)
            x[1] = x[0] ^ x[1]
        a, b = ksched[i]
        x[0] = x[0] + a
        x[1] = x[1] + b + np.uint32(i + 1)
    return x[0], x[1]


def _fixed_sample_indices(L_Q: int, L_K: int, U_part: int) -> np.ndarray:
    with np.errstate(over="ignore"):
        b1, b2 = _threefry2x32_np(np.uint32(0), np.uint32(42),
                                  np.zeros(2, np.uint32),
                                  np.arange(2, dtype=np.uint32))
        n = L_Q * U_part
        hi_cnt = np.zeros(n, np.uint32)
        lo_cnt = np.arange(n, dtype=np.uint32)
        ha, hb = _threefry2x32_np(b1[0], b2[0], hi_cnt, lo_cnt)
        la, lb = _threefry2x32_np(b1[1], b2[1], hi_cnt, lo_cnt)
        higher_bits, lower_bits = ha ^ hb, la ^ lb
        span = np.uint32(L_K)
        mult = np.uint32((2 ** 16) % L_K)
        mult = np.uint32((int(mult) * int(mult)) % L_K)
        offset = ((higher_bits % span) * mult + lower_bits % span) % span
    return offset.astype(np.int64).reshape(L_Q, U_part)


@functools.lru_cache(maxsize=None)
def _sample_count_matrix_T(L_Q: int, L_K: int, U_part: int):
    idx_np = _fixed_sample_indices(L_Q, L_K, U_part)
    ct = np.zeros((L_K, L_Q), np.float32)
    np.add.at(ct, (idx_np.reshape(-1), np.repeat(np.arange(L_Q), U_part)), 1.0)
    madd = np.where(ct > 0, np.float32(0.0), _NEG).astype(np.float32)
    return ct, madd


_sample_count_matrix_T(2048, 2048, 40)


def _body(u, u_pad, L_K, D, scale, q_ref, k_ref, v_ref, ct_ref, madd_ref,
          o_ref):
    L_Q = q_ref.shape[1]
    nh = q_ref.shape[2] // D
    Qs = [q_ref[0, :, h * D:(h + 1) * D] for h in range(nh)]
    Ks = [k_ref[0, :, h * D:(h + 1) * D] for h in range(nh)]
    Vs = [v_ref[0, :, h * D:(h + 1) * D] for h in range(nh)]

    CH = min(512, L_Q)
    n_ch = L_Q // CH
    rows = [[] for _ in range(nh)]
    for ci in range(n_ch):
        cct = ct_ref[:, ci * CH:(ci + 1) * CH]
        ma = madd_ref[:, ci * CH:(ci + 1) * CH]
        for h in range(nh):
            qc = Qs[h][ci * CH:(ci + 1) * CH, :]
            qkt = jax.lax.dot_general(Ks[h], qc, (((1,), (1,)), ((), ())),
                                      preferred_element_type=jnp.float32)
            smax = jnp.max(qkt + ma, axis=0)
            ssum = jnp.sum(qkt * cct, axis=0)
            m = smax - ssum * np.float32(1.0 / L_K)
            rows[h].append(m.reshape(1, CH))
    for h in range(nh):
        _one_head(u, u_pad, L_K, D, scale, h, rows[h], Qs[h], Ks[h], Vs[h],
                  o_ref)


def _one_head(u, u_pad, L_K, D, scale, h, rows, Q, K, V, o_ref):
    L_Q = o_ref.shape[1]
    n_ch = len(rows)
    M = jnp.concatenate(rows, axis=1) if n_ch > 1 else rows[0]

    lane_idx = jax.lax.broadcasted_iota(jnp.int32, (1, L_Q), 1)
    rank = jnp.zeros((1, L_Q), jnp.float32)
    RC = min(512, L_Q)
    ones8r = jnp.full((8, RC), np.float32(1.0), jnp.float32)
    eye = (jax.lax.broadcasted_iota(jnp.int32, (RC, RC), 0)
           == jax.lax.broadcasted_iota(jnp.int32, (RC, RC), 1)
           ).astype(jnp.float32)
    for rc in range(L_Q // RC):
        mrow = jax.lax.dot_general(
            eye, M[:, rc * RC:(rc + 1) * RC], (((1,), (1,)), ((), ())),
            preferred_element_type=jnp.float32)
        rowi = (jax.lax.broadcasted_iota(jnp.int32, (RC, 1), 0)
                + np.int32(rc * RC))
        gt = mrow > M
        eq = mrow == M
        tri = rowi < lane_idx
        cond = jnp.logical_or(gt, jnp.logical_and(eq, tri))
        condf = jnp.where(cond, np.float32(1.0), np.float32(0.0))
        rank = rank + jnp.sum(condf, axis=0, keepdims=True)

    ranki = rank.astype(jnp.int32)
    uio = jax.lax.broadcasted_iota(jnp.int32, (u_pad, 1), 0)
    R = ((ranki == uio) & (uio < np.int32(u))).astype(jnp.float32)

    Qr = jax.lax.dot_general(R, Q, (((1,), (0,)), ((), ())),
                             preferred_element_type=jnp.float32)
    scores = jax.lax.dot_general(Qr, K, (((1,), (1,)), ((), ())),
                                 preferred_element_type=jnp.float32)
    scores = scores * np.float32(scale)
    smax2 = jnp.max(scores, axis=1, keepdims=True)
    e = jnp.exp(scores - smax2)
    attn = e / jnp.sum(e, axis=1, keepdims=True)
    upd = jax.lax.dot_general(attn, V, (((1,), (0,)), ((), ())),
                              preferred_element_type=jnp.float32)
    mean_v = jnp.sum(V, axis=0, keepdims=True) * np.float32(1.0 / L_K)
    delta = upd - mean_v
    ctx = jax.lax.dot_general(R, delta, (((0,), (0,)), ((), ())),
                              preferred_element_type=jnp.float32)
    o_ref[0, :, h * D:(h + 1) * D] = ctx + mean_v


def kernel(queries, keys, values):
    B, L_Q, H, D = queries.shape
    L_K = keys.shape[1]
    U_part = min(_FACTOR * int(np.ceil(np.log(L_K))), L_K)
    u = min(_FACTOR * int(np.ceil(np.log(L_Q))), L_Q)
    u_pad = -(-u // 8) * 8
    scale = 1.0 / math.sqrt(D)
    ct, madd = _sample_count_matrix_T(L_Q, L_K, U_part)

    HPB = max(1, 128 // D)
    ng = H // HPB
    q3 = queries.reshape(B, L_Q, H * D)
    k3 = keys.reshape(B, L_K, H * D)
    v3 = values.reshape(B, L_K, H * D)
    bh = pl.BlockSpec((1, L_Q, HPB * D), lambda i: (i // ng, 0, i % ng))
    out = pl.pallas_call(
        functools.partial(_body, u, u_pad, L_K, D, scale),
        grid=(B * ng,),
        in_specs=[
            bh,
            bh,
            bh,
            pl.BlockSpec((L_K, L_Q), lambda i: (0, 0)),
            pl.BlockSpec((L_K, L_Q), lambda i: (0, 0)),
        ],
        out_specs=bh,
        out_shape=jax.ShapeDtypeStruct((B, L_Q, H * D), jnp.float32),
        compiler_params=pltpu.CompilerParams(
            dimension_semantics=("arbitrary",),
        ),
    )(q3, k3, v3, ct, madd)

    return out.reshape(B, L_Q, H, D)
